# Initial kernel scaffold; baseline (speedup 1.0000x reference)
import jax
import jax.numpy as jnp
from jax.experimental import pallas as pl


def kernel(x, pos_table):
    raise NotImplementedError("write your pallas kernel here")


# speedup vs baseline: 1.7237x; 1.7237x over previous
import jax
import jax.numpy as jnp
from jax.experimental import pallas as pl

_SEQ_BLK = 512


def _add_kernel(x_ref, pos_ref, o_ref):
    o_ref[...] = x_ref[...] + pos_ref[...]


def kernel(x, pos_table):
    batch, seq, dim = x.shape
    blk = min(_SEQ_BLK, seq)
    grid = (seq // blk,)
    return pl.pallas_call(
        _add_kernel,
        grid=grid,
        in_specs=[
            pl.BlockSpec((batch, blk, dim), lambda i: (0, i, 0)),
            pl.BlockSpec((blk, dim), lambda i: (i, 0)),
        ],
        out_specs=pl.BlockSpec((batch, blk, dim), lambda i: (0, i, 0)),
        out_shape=jax.ShapeDtypeStruct((batch, seq, dim), x.dtype),
    )(x, pos_table)


# speedup vs baseline: 1.7388x; 1.0087x over previous
import jax
import jax.numpy as jnp
from jax.experimental import pallas as pl

_SEQ_BLK = 2048


def _add_kernel(x_ref, pos_ref, o_ref):
    o_ref[...] = x_ref[...] + pos_ref[...]


def kernel(x, pos_table):
    batch, seq, dim = x.shape
    blk = min(_SEQ_BLK, seq)
    grid = (seq // blk, batch)
    return pl.pallas_call(
        _add_kernel,
        grid=grid,
        in_specs=[
            pl.BlockSpec((1, blk, dim), lambda i, b: (b, i, 0)),
            pl.BlockSpec((blk, dim), lambda i, b: (i, 0)),
        ],
        out_specs=pl.BlockSpec((1, blk, dim), lambda i, b: (b, i, 0)),
        out_shape=jax.ShapeDtypeStruct((batch, seq, dim), x.dtype),
    )(x, pos_table)
